# baseline (speedup 1.0000x reference)
import functools

import jax
import jax.numpy as jnp
from jax import lax
from jax.experimental import pallas as pl
from jax.experimental.pallas import tpu as pltpu
from jax.experimental.pallas import tpu_sc as plsc

B = 128
V = 100000
NC = 2
NS = 16
SH = 6272
VP = NS * SH
NVR = SH // 16
SH2 = SH // 4
NVR2 = SH2 // 16
NCH = SH // 128
RPC = B // NC
LAST = NS - 1
T15 = V - LAST * SH
BUF = VP + 128
MSB = -2147483648
MAXI = 2147483647
NEG_INF = float("-inf")


def _keys_from_f32(x):
    u = plsc.bitcast(x, jnp.int32)
    return jnp.where(u < 0, u, MAXI - u)


def _f32_from_keys(k):
    u = jnp.where(k < 0, k, MAXI - k)
    return plsc.bitcast(u, jnp.float32)


def _lane(vec, lane):
    iota = lax.iota(jnp.int32, 16)
    return jnp.sum(jnp.where(iota == lane, vec, jnp.zeros_like(vec)))


def _body(logits, topk, topp, minp, o_sel, o_tkp, o_idx, o_srt,
          tf0, tk0, tkc, tic, tkS, tiS, pS, hist, offs, hmat, tots, cb, cbf,
          t16, f16, tkv, tpv, mpv, selb, semk, semi,
          sk0, si0, sk1, si1, shh, shf, shf2):
    cid = lax.axis_index("c")
    tid = lax.axis_index("s")
    iota = lax.iota(jnp.int32, 16)
    ones = jnp.ones((16,), jnp.int32)
    zeros = jnp.zeros((16,), jnp.int32)
    fzeros = jnp.zeros((16,), jnp.float32)
    last = tid == LAST
    notlast = jnp.logical_not(last)
    rn = jnp.where(last, T15, SH)
    tbase = pl.multiple_of(tid * SH, 128)
    iotaN = iota * NVR
    iotaN2 = iota * NVR2
    ib256 = iota * 256

    pltpu.sync_copy(topk, tkv)
    pltpu.sync_copy(topp, tpv)
    pltpu.sync_copy(minp, mpv)

    srcs = [(sk0, si0), (sk1, si1)]

    def copy_share_to_hbm(src_ref, dst, row):
        pltpu.sync_copy(src_ref, dst.at[row, pl.ds(tbase, SH)])

    def row_body(rl, carry):
        row = cid * RPC + rl

        pltpu.sync_copy(logits.at[row, pl.ds(tbase, SH)], tf0)

        def zl0(j, c):
            hist[pl.ds(j * 16, 16)] = zeros
            return c
        lax.fori_loop(0, 1024, zl0, 0)

        def tx(v, c):
            pad = jnp.full((16,), -1, jnp.int32)
            for hh in range(4):
                gi = iotaN2 + v + hh * SH2
                x = plsc.load_gather(tf0, [gi])
                k = jnp.where(gi < rn, _keys_from_f32(x), pad)
                plsc.store_scatter(tk0, [gi], k)
                plsc.addupdate_scatter(hist, [hh * 4096 + ib256 + (k & 255)],
                                       ones)
            return c
        lax.fori_loop(0, NVR2, tx, 0)

        for p in range(4):
            if p == 0:
                ksrc = tk0
            else:
                skA, siA = srcs[(p + 1) % 2]
                pltpu.sync_copy(skA.at[pl.ds(tbase, SH)], tkc)
                pltpu.sync_copy(siA.at[pl.ds(tbase, SH)], tic)
                ksrc = tkc
            dK, dI = srcs[p % 2]
            shv = jnp.full((16,), p * 8, jnp.int32)

            if p > 0:
                def zloop(j, c):
                    hist[pl.ds(j * 16, 16)] = zeros
                    return c
                lax.fori_loop(0, 1024, zloop, 0)

                def hloop(v, c, ksrc=ksrc, shv=shv):
                    for hh in range(4):
                        gi = iotaN2 + v + hh * SH2
                        x = plsc.load_gather(ksrc, [gi])
                        d = lax.shift_right_logical(x, shv) & 255
                        plsc.addupdate_scatter(hist, [hh * 4096 + ib256 + d],
                                               ones)
                    return c
                lax.fori_loop(0, NVR2, hloop, 0)

            def tloop(j, c):
                s = zeros
                for hh in range(4):
                    for l in range(NS):
                        s = s + hist[pl.ds(hh * 4096 + l * 256 + j * 16, 16)]
                tots[pl.ds(j * 16, 16)] = s
                return c
            lax.fori_loop(0, 16, tloop, 0)
            pltpu.sync_copy(tots, shh.at[pl.ds(pl.multiple_of(tid * 256, 256), 256)])
            plsc.subcore_barrier()
            pltpu.sync_copy(shh, hmat)

            def oloop(j, gcarry):
                su = zeros
                pre = zeros
                for t2 in range(NS):
                    r = hmat[pl.ds(t2 * 256 + j * 16, 16)]
                    su = su + r
                    pre = pre + jnp.where(t2 < tid, r, zeros)
                gb = plsc.cumsum(su) - su + gcarry
                tbv = gb + pre
                acc = tbv
                for hh in range(4):
                    for l in range(NS):
                        offs[pl.ds(hh * 4096 + l * 256 + j * 16, 16)] = acc
                        acc = acc + hist[pl.ds(hh * 4096 + l * 256 + j * 16, 16)]
                return gcarry + jnp.sum(su)
            lax.fori_loop(0, 16, oloop, jnp.int32(0))

            def ploop(jc, c, ksrc=ksrc, shv=shv, p=p, dK=dK, dI=dI):
                if p == 0:
                    tsrc = None
                else:
                    tsrc = tic
                for q in range(2):
                    v = jc * 2 + q
                    for hh in range(4):
                        gi = iotaN2 + v + hh * SH2
                        x = plsc.load_gather(ksrc, [gi])
                        d = lax.shift_right_logical(x, shv) & 255
                        ib = hh * 4096 + ib256 + d
                        if p == 0:
                            xi = tbase + gi
                        else:
                            xi = plsc.load_gather(tsrc, [gi])
                        cur = plsc.load_gather(offs, [ib])
                        plsc.addupdate_scatter(offs, [ib], ones)
                        sl = pl.ds(q * 64 + hh * 16, 16)
                        tkS[jc, sl] = x
                        tiS[jc, sl] = xi
                        pS[jc, sl] = cur
                dk = pltpu.async_copy(tkS.at[jc], dK.at[pS.at[jc]], semk)
                di = pltpu.async_copy(tiS.at[jc], dI.at[pS.at[jc]], semi)
                dk.wait()
                di.wait()
                return c
            lax.fori_loop(0, NCH, ploop, 0)
            plsc.subcore_barrier()

        rb16 = pl.multiple_of((row >> 4) << 4, 16)
        rlane = row - rb16
        topk_r = _lane(tkv[pl.ds(rb16, 16)], rlane)
        topp_r = _lane(tpv[pl.ds(rb16, 16)], rlane)
        minp_r = _lane(mpv[pl.ds(rb16, 16)], rlane)
        valid = topk_r >= 1
        pos = jnp.where(valid, topk_r - 1, 0)
        b8 = pl.multiple_of((pos >> 3) << 3, 8)
        pltpu.sync_copy(sk1.at[pl.ds(b8, 16)], t16)
        kth_k = _lane(t16[...], pos - b8)
        kth_s = jnp.where(valid, kth_k ^ MSB, MAXI)

        def kloop(v, c):
            x = tk0[pl.ds(v * 16, 16)]
            g = v * 16 + iota
            m = jnp.logical_and((x ^ MSB) > kth_s, g < rn)
            return c + jnp.sum(m.astype(jnp.int32))
        cm = lax.fori_loop(0, NVR, kloop, jnp.int32(0))
        t16[...] = zeros + cm
        pltpu.sync_copy(t16, shh.at[pl.ds(pl.multiple_of(tid * 16, 16), 16)])

        pltpu.sync_copy(sk1.at[pl.ds(0, 16)], t16)
        maxv = _lane(_f32_from_keys(t16[...]), 0)
        pltpu.sync_copy(sk1.at[pl.ds(tbase, SH)], tkc)

        def dloop(v, acc):
            k = tkc[pl.ds(v * 16, 16)]
            f = _f32_from_keys(k)
            g = tbase + v * 16 + iota
            m = jnp.logical_and((k ^ MSB) <= kth_s, g < V)
            e = jnp.where(m, jnp.exp(f - maxv), jnp.float32(0.0))
            tic[pl.ds(v * 16, 16)] = plsc.bitcast(e, jnp.int32)
            return acc + jnp.sum(e)
        esum = lax.fori_loop(0, NVR, dloop, jnp.float32(0.0))
        f16[...] = fzeros + esum
        pltpu.sync_copy(f16, shf.at[pl.ds(pl.multiple_of(tid * 16, 16), 16)])
        plsc.subcore_barrier()

        pltpu.sync_copy(shh.at[pl.ds(0, 256)], cb)
        pltpu.sync_copy(shf, cbf)
        pref = jnp.int32(0)
        totm = jnp.int32(0)
        for t2 in range(NS):
            cv = _lane(cb[pl.ds(t2 * 16, 16)], 0)
            totm = totm + cv
            pref = pref + jnp.where(t2 < tid, cv, jnp.int32(0))
        kept = V - totm
        den = jnp.float32(0.0)
        for t2 in range(NS):
            ev = _lane(cbf[pl.ds(t2 * 16, 16)], 0)
            den = den + ev
        rden = _lane((fzeros + 1.0) / (fzeros + den), 0)
        prefp = jnp.float32(0.0)
        myp = jnp.float32(0.0)
        totp = jnp.float32(0.0)
        for t2 in range(NS):
            ev = _lane(cbf[pl.ds(t2 * 16, 16)], 0)
            pv = ev * rden
            totp = totp + pv
            prefp = prefp + jnp.where(t2 < tid, pv, jnp.float32(0.0))
            myp = myp + jnp.where(t2 == tid, pv, jnp.float32(0.0))
        qual = jnp.logical_and(prefp < topp_r, prefp + myp >= topp_r)
        qual0 = jnp.logical_and(tid == 0, topp_r <= 0.0)
        nobody = jnp.logical_and(tid == 0, totp < topp_r)
        writer = jnp.logical_or(jnp.logical_or(qual, qual0), nobody)

        def cloop(v, run):
            x = tk0[pl.ds(v * 16, 16)]
            g = v * 16 + iota
            m = jnp.logical_and((x ^ MSB) > kth_s, g < rn)
            mi = m.astype(jnp.int32)
            csum = plsc.cumsum(mi)
            pp = run + csum - 1
            pf = jnp.where(m, pp, VP + iota)
            jc = v >> 3
            sl = pl.ds((v - (jc << 3)) * 16, 16)
            pS[jc, sl] = pf
            tiS[jc, sl] = tbase + g
            return run + jnp.sum(mi)
        lax.fori_loop(0, NVR, cloop, kept + pref)

        def cdma(jc, c):
            pltpu.async_copy(tiS.at[jc], si1.at[pS.at[jc]], semi).wait()
            return c
        lax.fori_loop(0, NCH, cdma, 0)

        def floop(v, fcarry):
            run, found, thr = fcarry
            e = plsc.bitcast(tic[pl.ds(v * 16, 16)], jnp.float32)
            pvec = e * rden
            cum = plsc.cumsum(pvec) + run
            condv = jnp.logical_and(cum >= topp_r, pvec > 0.0)
            anyv = jnp.sum(condv.astype(jnp.int32)) > 0
            tv = jnp.max(jnp.where(condv, pvec, NEG_INF))
            hit = jnp.logical_and(anyv, found == 0)
            thr = jnp.where(hit, tv, thr)
            found = jnp.where(anyv, jnp.int32(1), found)
            return (run + jnp.sum(pvec), found, thr)
        _, fnd, thr = lax.fori_loop(
            0, NVR, floop, (prefp, jnp.int32(0), jnp.float32(0.0)))
        thrv = jnp.where(fnd == 1, thr, jnp.float32(0.0))
        f16[...] = fzeros + thrv

        @pl.when(writer)
        def _():
            pltpu.sync_copy(f16, shf2)
        plsc.subcore_barrier()
        pltpu.sync_copy(shf2, f16)
        thrg = _lane(f16[...], 0)
        comb = jnp.maximum(rden * minp_r, thrg)

        def e1(v, c):
            k = tkc[pl.ds(v * 16, 16)]
            f = _f32_from_keys(k)
            e = plsc.bitcast(tic[pl.ds(v * 16, 16)], jnp.float32)
            keep = e * rden >= comb
            tf0[pl.ds(v * 16, 16)] = jnp.where(keep, f, NEG_INF)
            return c
        lax.fori_loop(0, NVR, e1, 0)
        copy_share_to_hbm(tf0, o_srt, row)

        pltpu.sync_copy(si1.at[pl.ds(tbase, SH)],
                        o_idx.at[row, pl.ds(tbase, SH)])

        def e2(v, c):
            k = tk0[pl.ds(v * 16, 16)]
            f = _f32_from_keys(k)
            ks = k ^ MSB
            pp = jnp.exp(f - maxv) * rden
            keep = jnp.logical_and(ks <= kth_s, pp >= comb)
            tf0[pl.ds(v * 16, 16)] = jnp.where(keep, f, NEG_INF)
            return c
        lax.fori_loop(0, NVR, e2, 0)
        copy_share_to_hbm(tf0, o_tkp, row)

        @pl.when(tid == 0)
        def _():
            pltpu.sync_copy(si1.at[pl.ds(0, 16)], t16)
            sv = _lane(t16[...], 0)
            sb = pl.multiple_of((rl >> 4) << 4, 16)
            old = selb[pl.ds(sb, 16)]
            selb[pl.ds(sb, 16)] = jnp.where(iota == rl - sb, zeros + sv, old)
        return carry

    lax.fori_loop(0, RPC, row_body, 0)

    @pl.when(tid == 0)
    def _():
        pltpu.sync_copy(selb, o_sel.at[pl.ds(pl.multiple_of(cid * RPC, RPC), RPC)])


_mesh = plsc.VectorSubcoreMesh(core_axis_name="c", subcore_axis_name="s")

_sc_call = pl.kernel(
    _body,
    out_type=(
        jax.ShapeDtypeStruct((B,), jnp.int32),
        jax.ShapeDtypeStruct((B, VP), jnp.float32),
        jax.ShapeDtypeStruct((B, VP), jnp.int32),
        jax.ShapeDtypeStruct((B, VP), jnp.float32),
    ),
    mesh=_mesh,
    compiler_params=pltpu.CompilerParams(needs_layout_passes=False),
    scratch_types=[
        pltpu.VMEM((SH,), jnp.float32),
        pltpu.VMEM((SH,), jnp.int32),
        pltpu.VMEM((SH,), jnp.int32),
        pltpu.VMEM((SH,), jnp.int32),
        pltpu.VMEM((NCH, 128), jnp.int32),
        pltpu.VMEM((NCH, 128), jnp.int32),
        pltpu.VMEM((NCH, 128), jnp.int32),
        pltpu.VMEM((16384,), jnp.int32),
        pltpu.VMEM((16384,), jnp.int32),
        pltpu.VMEM((4096,), jnp.int32),
        pltpu.VMEM((256,), jnp.int32),
        pltpu.VMEM((256,), jnp.int32),
        pltpu.VMEM((256,), jnp.float32),
        pltpu.VMEM((16,), jnp.int32),
        pltpu.VMEM((16,), jnp.float32),
        pltpu.VMEM((B,), jnp.int32),
        pltpu.VMEM((B,), jnp.float32),
        pltpu.VMEM((B,), jnp.float32),
        pltpu.VMEM((RPC,), jnp.int32),
        pltpu.SemaphoreType.DMA,
        pltpu.SemaphoreType.DMA,
        pltpu.VMEM_SHARED((BUF,), jnp.int32),
        pltpu.VMEM_SHARED((BUF,), jnp.int32),
        pltpu.VMEM_SHARED((BUF,), jnp.int32),
        pltpu.VMEM_SHARED((BUF,), jnp.int32),
        pltpu.VMEM_SHARED((4096,), jnp.int32),
        pltpu.VMEM_SHARED((256,), jnp.float32),
        pltpu.VMEM_SHARED((16,), jnp.float32),
    ],
)


def kernel(logits, topK, topP, q, minPs, eps, isNeedLogits, topKGuess,
           ksMAX, inputIsLogits, isNeedSampleResult):
    del q, eps, isNeedLogits, topKGuess, ksMAX, inputIsLogits
    del isNeedSampleResult
    lp = jnp.pad(logits.astype(jnp.float32), ((0, 0), (0, VP - V)))
    sel, tkp, idx, srt = _sc_call(
        lp,
        topK.astype(jnp.int32),
        topP.astype(jnp.float32),
        minPs.astype(jnp.float32),
    )
    return sel, tkp[:, :V], idx[:, :V], srt[:, :V]


# speedup vs baseline: 1.3035x; 1.3035x over previous
import functools

import jax
import jax.numpy as jnp
from jax import lax
from jax.experimental import pallas as pl
from jax.experimental.pallas import tpu as pltpu
from jax.experimental.pallas import tpu_sc as plsc

B = 128
V = 100000
NC = 2
NS = 16
SH = 6272
VP = NS * SH
NVR = SH // 16
SH2 = SH // 2
NVR2 = SH2 // 16
NCH = SH // 128
RPC = B // NC
LAST = NS - 1
T15 = V - LAST * SH
BUF = VP + 128
MSB = -2147483648
MAXI = 2147483647
NEG_INF = float("-inf")


def _keys_from_f32(x):
    u = plsc.bitcast(x, jnp.int32)
    return jnp.where(u < 0, u, MAXI - u)


def _f32_from_keys(k):
    u = jnp.where(k < 0, k, MAXI - k)
    return plsc.bitcast(u, jnp.float32)


def _lane(vec, lane):
    iota = lax.iota(jnp.int32, 16)
    return jnp.sum(jnp.where(iota == lane, vec, jnp.zeros_like(vec)))


def _body(logits, topk, topp, minp, o_sel, o_tkp, o_idx, o_srt,
          tf0, tk0, tkc, tic, tkS, tiS, pS, hist, offs, hmat, tots, cb, cbf,
          t16, f16, tkv, tpv, mpv, selb, semk, semi,
          sk0, si0, sk1, si1, shh, shf, shf2):
    cid = lax.axis_index("c")
    tid = lax.axis_index("s")
    iota = lax.iota(jnp.int32, 16)
    ones = jnp.ones((16,), jnp.int32)
    zeros = jnp.zeros((16,), jnp.int32)
    fzeros = jnp.zeros((16,), jnp.float32)
    last = tid == LAST
    notlast = jnp.logical_not(last)
    rn = jnp.where(last, T15, SH)
    tbase = pl.multiple_of(tid * SH, 128)
    iotaN = iota * NVR
    iotaN2 = iota * NVR2
    ib256 = iota * 256

    pltpu.sync_copy(topk, tkv)
    pltpu.sync_copy(topp, tpv)
    pltpu.sync_copy(minp, mpv)

    srcs = [(sk0, si0), (sk1, si1)]

    def copy_share_to_hbm(src_ref, dst, row):
        pltpu.sync_copy(src_ref, dst.at[row, pl.ds(tbase, SH)])

    def row_body(rl, carry):
        row = cid * RPC + rl

        pltpu.sync_copy(logits.at[row, pl.ds(tbase, SH)], tf0)

        def zl0(j, c):
            hist[pl.ds(j * 16, 16)] = zeros
            return c
        lax.fori_loop(0, 512, zl0, 0)

        def tx(v, c):
            pad = jnp.full((16,), -1, jnp.int32)
            giA = iotaN2 + v
            giB = giA + SH2
            xA = plsc.load_gather(tf0, [giA])
            xB = plsc.load_gather(tf0, [giB])
            kA = jnp.where(giA < rn, _keys_from_f32(xA), pad)
            kB = jnp.where(giB < rn, _keys_from_f32(xB), pad)
            plsc.store_scatter(tk0, [giA], kA)
            plsc.store_scatter(tk0, [giB], kB)
            plsc.addupdate_scatter(hist, [ib256 + (kA & 255)], ones)
            plsc.addupdate_scatter(hist, [4096 + ib256 + (kB & 255)], ones)
            return c
        lax.fori_loop(0, NVR2, tx, 0)

        for p in range(4):
            if p == 0:
                ksrc = tk0
            else:
                skA, siA = srcs[(p + 1) % 2]
                pltpu.sync_copy(skA.at[pl.ds(tbase, SH)], tkc)
                pltpu.sync_copy(siA.at[pl.ds(tbase, SH)], tic)
                ksrc = tkc
            dK, dI = srcs[p % 2]
            shv = jnp.full((16,), p * 8, jnp.int32)

            if p > 0:
                def zloop(j, c):
                    hist[pl.ds(j * 16, 16)] = zeros
                    return c
                lax.fori_loop(0, 512, zloop, 0)

                def hloop(v, c, ksrc=ksrc, shv=shv):
                    giA = iotaN2 + v
                    giB = giA + SH2
                    xA = plsc.load_gather(ksrc, [giA])
                    xB = plsc.load_gather(ksrc, [giB])
                    dA = lax.shift_right_logical(xA, shv) & 255
                    dB = lax.shift_right_logical(xB, shv) & 255
                    plsc.addupdate_scatter(hist, [ib256 + dA], ones)
                    plsc.addupdate_scatter(hist, [4096 + ib256 + dB], ones)
                    return c
                lax.fori_loop(0, NVR2, hloop, 0)

            def tloop(j, c):
                s = zeros
                for hh in range(2):
                    for l in range(NS):
                        s = s + hist[pl.ds(hh * 4096 + l * 256 + j * 16, 16)]
                tots[pl.ds(j * 16, 16)] = s
                return c
            lax.fori_loop(0, 16, tloop, 0)
            pltpu.sync_copy(tots, shh.at[pl.ds(pl.multiple_of(tid * 256, 256), 256)])
            plsc.subcore_barrier()
            pltpu.sync_copy(shh, hmat)

            def oloop(j, gcarry):
                su = zeros
                pre = zeros
                for t2 in range(NS):
                    r = hmat[pl.ds(t2 * 256 + j * 16, 16)]
                    su = su + r
                    pre = pre + jnp.where(t2 < tid, r, zeros)
                gb = plsc.cumsum(su) - su + gcarry
                tbv = gb + pre
                acc = tbv
                for hh in range(2):
                    for l in range(NS):
                        offs[pl.ds(hh * 4096 + l * 256 + j * 16, 16)] = acc
                        acc = acc + hist[pl.ds(hh * 4096 + l * 256 + j * 16, 16)]
                return gcarry + jnp.sum(su)
            lax.fori_loop(0, 16, oloop, jnp.int32(0))

            def ploop(jc, c, ksrc=ksrc, shv=shv, p=p, dK=dK, dI=dI):
                if p == 0:
                    tsrc = None
                else:
                    tsrc = tic
                for q in range(4):
                    v = jc * 4 + q
                    giA = iotaN2 + v
                    giB = giA + SH2
                    xA = plsc.load_gather(ksrc, [giA])
                    xB = plsc.load_gather(ksrc, [giB])
                    dA = lax.shift_right_logical(xA, shv) & 255
                    dB = lax.shift_right_logical(xB, shv) & 255
                    ibA = ib256 + dA
                    ibB = 4096 + ib256 + dB
                    if p == 0:
                        xiA = tbase + giA
                        xiB = tbase + giB
                    else:
                        xiA = plsc.load_gather(tsrc, [giA])
                        xiB = plsc.load_gather(tsrc, [giB])
                    curA = plsc.load_gather(offs, [ibA])
                    curB = plsc.load_gather(offs, [ibB])
                    plsc.addupdate_scatter(offs, [ibA], ones)
                    plsc.addupdate_scatter(offs, [ibB], ones)
                    slA = pl.ds(q * 32, 16)
                    slB = pl.ds(q * 32 + 16, 16)
                    tkS[jc, slA] = xA
                    tkS[jc, slB] = xB
                    tiS[jc, slA] = xiA
                    tiS[jc, slB] = xiB
                    pS[jc, slA] = curA
                    pS[jc, slB] = curB
                dk = pltpu.async_copy(tkS.at[jc], dK.at[pS.at[jc]], semk)
                di = pltpu.async_copy(tiS.at[jc], dI.at[pS.at[jc]], semi)
                dk.wait()
                di.wait()
                return c
            lax.fori_loop(0, NCH, ploop, 0)
            plsc.subcore_barrier()

        rb16 = pl.multiple_of((row >> 4) << 4, 16)
        rlane = row - rb16
        topk_r = _lane(tkv[pl.ds(rb16, 16)], rlane)
        topp_r = _lane(tpv[pl.ds(rb16, 16)], rlane)
        minp_r = _lane(mpv[pl.ds(rb16, 16)], rlane)
        valid = topk_r >= 1
        pos = jnp.where(valid, topk_r - 1, 0)
        b8 = pl.multiple_of((pos >> 3) << 3, 8)
        pltpu.sync_copy(sk1.at[pl.ds(b8, 16)], t16)
        kth_k = _lane(t16[...], pos - b8)
        kth_s = jnp.where(valid, kth_k ^ MSB, MAXI)

        def kloop(v, c):
            x = tk0[pl.ds(v * 16, 16)]
            g = v * 16 + iota
            m = jnp.logical_and((x ^ MSB) > kth_s, g < rn)
            return c + jnp.sum(m.astype(jnp.int32))
        cm = lax.fori_loop(0, NVR, kloop, jnp.int32(0))
        t16[...] = zeros + cm
        pltpu.sync_copy(t16, shh.at[pl.ds(pl.multiple_of(tid * 16, 16), 16)])

        pltpu.sync_copy(sk1.at[pl.ds(0, 16)], t16)
        maxv = _lane(_f32_from_keys(t16[...]), 0)
        pltpu.sync_copy(sk1.at[pl.ds(tbase, SH)], tkc)

        def dloop(v, acc):
            k = tkc[pl.ds(v * 16, 16)]
            f = _f32_from_keys(k)
            g = tbase + v * 16 + iota
            m = jnp.logical_and((k ^ MSB) <= kth_s, g < V)
            e = jnp.where(m, jnp.exp(f - maxv), jnp.float32(0.0))
            tic[pl.ds(v * 16, 16)] = plsc.bitcast(e, jnp.int32)
            return acc + jnp.sum(e)
        esum = lax.fori_loop(0, NVR, dloop, jnp.float32(0.0))
        f16[...] = fzeros + esum
        pltpu.sync_copy(f16, shf.at[pl.ds(pl.multiple_of(tid * 16, 16), 16)])
        plsc.subcore_barrier()

        pltpu.sync_copy(shh.at[pl.ds(0, 256)], cb)
        pltpu.sync_copy(shf, cbf)
        pref = jnp.int32(0)
        totm = jnp.int32(0)
        for t2 in range(NS):
            cv = _lane(cb[pl.ds(t2 * 16, 16)], 0)
            totm = totm + cv
            pref = pref + jnp.where(t2 < tid, cv, jnp.int32(0))
        kept = V - totm
        den = jnp.float32(0.0)
        for t2 in range(NS):
            ev = _lane(cbf[pl.ds(t2 * 16, 16)], 0)
            den = den + ev
        rden = _lane((fzeros + 1.0) / (fzeros + den), 0)
        prefp = jnp.float32(0.0)
        myp = jnp.float32(0.0)
        totp = jnp.float32(0.0)
        for t2 in range(NS):
            ev = _lane(cbf[pl.ds(t2 * 16, 16)], 0)
            pv = ev * rden
            totp = totp + pv
            prefp = prefp + jnp.where(t2 < tid, pv, jnp.float32(0.0))
            myp = myp + jnp.where(t2 == tid, pv, jnp.float32(0.0))
        qual = jnp.logical_and(prefp < topp_r, prefp + myp >= topp_r)
        qual0 = jnp.logical_and(tid == 0, topp_r <= 0.0)
        nobody = jnp.logical_and(tid == 0, totp < topp_r)
        writer = jnp.logical_or(jnp.logical_or(qual, qual0), nobody)

        def cloop(v, run):
            x = tk0[pl.ds(v * 16, 16)]
            g = v * 16 + iota
            m = jnp.logical_and((x ^ MSB) > kth_s, g < rn)
            mi = m.astype(jnp.int32)
            csum = plsc.cumsum(mi)
            pp = run + csum - 1
            pf = jnp.where(m, pp, VP + iota)
            jc = v >> 3
            sl = pl.ds((v - (jc << 3)) * 16, 16)
            pS[jc, sl] = pf
            tiS[jc, sl] = tbase + g
            return run + jnp.sum(mi)
        lax.fori_loop(0, NVR, cloop, kept + pref)

        def cdma(jc, c):
            pltpu.async_copy(tiS.at[jc], si1.at[pS.at[jc]], semi).wait()
            return c
        lax.fori_loop(0, NCH, cdma, 0)

        def floop(v, fcarry):
            run, found, thr = fcarry
            e = plsc.bitcast(tic[pl.ds(v * 16, 16)], jnp.float32)
            pvec = e * rden
            cum = plsc.cumsum(pvec) + run
            condv = jnp.logical_and(cum >= topp_r, pvec > 0.0)
            anyv = jnp.sum(condv.astype(jnp.int32)) > 0
            tv = jnp.max(jnp.where(condv, pvec, NEG_INF))
            hit = jnp.logical_and(anyv, found == 0)
            thr = jnp.where(hit, tv, thr)
            found = jnp.where(anyv, jnp.int32(1), found)
            return (run + jnp.sum(pvec), found, thr)
        _, fnd, thr = lax.fori_loop(
            0, NVR, floop, (prefp, jnp.int32(0), jnp.float32(0.0)))
        thrv = jnp.where(fnd == 1, thr, jnp.float32(0.0))
        f16[...] = fzeros + thrv

        @pl.when(writer)
        def _():
            pltpu.sync_copy(f16, shf2)
        plsc.subcore_barrier()
        pltpu.sync_copy(shf2, f16)
        thrg = _lane(f16[...], 0)
        comb = jnp.maximum(rden * minp_r, thrg)

        def e1(v, c):
            k = tkc[pl.ds(v * 16, 16)]
            f = _f32_from_keys(k)
            e = plsc.bitcast(tic[pl.ds(v * 16, 16)], jnp.float32)
            keep = e * rden >= comb
            tf0[pl.ds(v * 16, 16)] = jnp.where(keep, f, NEG_INF)
            return c
        lax.fori_loop(0, NVR, e1, 0)
        copy_share_to_hbm(tf0, o_srt, row)

        pltpu.sync_copy(si1.at[pl.ds(tbase, SH)],
                        o_idx.at[row, pl.ds(tbase, SH)])

        def e2(v, c):
            k = tk0[pl.ds(v * 16, 16)]
            f = _f32_from_keys(k)
            ks = k ^ MSB
            pp = jnp.exp(f - maxv) * rden
            keep = jnp.logical_and(ks <= kth_s, pp >= comb)
            tf0[pl.ds(v * 16, 16)] = jnp.where(keep, f, NEG_INF)
            return c
        lax.fori_loop(0, NVR, e2, 0)
        copy_share_to_hbm(tf0, o_tkp, row)

        @pl.when(tid == 0)
        def _():
            pltpu.sync_copy(si1.at[pl.ds(0, 16)], t16)
            sv = _lane(t16[...], 0)
            sb = pl.multiple_of((rl >> 4) << 4, 16)
            old = selb[pl.ds(sb, 16)]
            selb[pl.ds(sb, 16)] = jnp.where(iota == rl - sb, zeros + sv, old)
        return carry

    lax.fori_loop(0, RPC, row_body, 0)

    @pl.when(tid == 0)
    def _():
        pltpu.sync_copy(selb, o_sel.at[pl.ds(pl.multiple_of(cid * RPC, RPC), RPC)])


_mesh = plsc.VectorSubcoreMesh(core_axis_name="c", subcore_axis_name="s")

_sc_call = pl.kernel(
    _body,
    out_type=(
        jax.ShapeDtypeStruct((B,), jnp.int32),
        jax.ShapeDtypeStruct((B, VP), jnp.float32),
        jax.ShapeDtypeStruct((B, VP), jnp.int32),
        jax.ShapeDtypeStruct((B, VP), jnp.float32),
    ),
    mesh=_mesh,
    compiler_params=pltpu.CompilerParams(needs_layout_passes=False),
    scratch_types=[
        pltpu.VMEM((SH,), jnp.float32),
        pltpu.VMEM((SH,), jnp.int32),
        pltpu.VMEM((SH,), jnp.int32),
        pltpu.VMEM((SH,), jnp.int32),
        pltpu.VMEM((NCH, 128), jnp.int32),
        pltpu.VMEM((NCH, 128), jnp.int32),
        pltpu.VMEM((NCH, 128), jnp.int32),
        pltpu.VMEM((8192,), jnp.int32),
        pltpu.VMEM((8192,), jnp.int32),
        pltpu.VMEM((4096,), jnp.int32),
        pltpu.VMEM((256,), jnp.int32),
        pltpu.VMEM((256,), jnp.int32),
        pltpu.VMEM((256,), jnp.float32),
        pltpu.VMEM((16,), jnp.int32),
        pltpu.VMEM((16,), jnp.float32),
        pltpu.VMEM((B,), jnp.int32),
        pltpu.VMEM((B,), jnp.float32),
        pltpu.VMEM((B,), jnp.float32),
        pltpu.VMEM((RPC,), jnp.int32),
        pltpu.SemaphoreType.DMA,
        pltpu.SemaphoreType.DMA,
        pltpu.VMEM_SHARED((BUF,), jnp.int32),
        pltpu.VMEM_SHARED((BUF,), jnp.int32),
        pltpu.VMEM_SHARED((BUF,), jnp.int32),
        pltpu.VMEM_SHARED((BUF,), jnp.int32),
        pltpu.VMEM_SHARED((4096,), jnp.int32),
        pltpu.VMEM_SHARED((256,), jnp.float32),
        pltpu.VMEM_SHARED((16,), jnp.float32),
    ],
)


def kernel(logits, topK, topP, q, minPs, eps, isNeedLogits, topKGuess,
           ksMAX, inputIsLogits, isNeedSampleResult):
    del q, eps, isNeedLogits, topKGuess, ksMAX, inputIsLogits
    del isNeedSampleResult
    lp = jnp.pad(logits.astype(jnp.float32), ((0, 0), (0, VP - V)))
    sel, tkp, idx, srt = _sc_call(
        lp,
        topK.astype(jnp.int32),
        topP.astype(jnp.float32),
        minPs.astype(jnp.float32),
    )
    return sel, tkp[:, :V], idx[:, :V], srt[:, :V]
